# Initial kernel scaffold; baseline (speedup 1.0000x reference)
import jax
import jax.numpy as jnp
from jax import lax
from jax.experimental import pallas as pl
from jax.experimental.pallas import tpu as pltpu
from jax.experimental.pallas import tpu_sc as plsc

B = 16384
D = 128
L = 16
NC = 2
NS = 16
NW = NC * NS
BPW = B // NW
C = 128
NCHUNK = BPW // C


def _compute_chunk(h_ref, r_ref, t_ref, part_ref, out_ref, out_base):
    def body(e, carry):
        acc = jnp.zeros((L,), jnp.float32)
        for j in range(D // L):
            hv = h_ref[e, pl.ds(j * L, L)]
            rv = r_ref[e, pl.ds(j * L, L)]
            tv = t_ref[e, pl.ds(j * L, L)]
            acc = acc + hv * rv * tv
        part_ref[pl.ds(e * L, L)] = acc
        return carry

    lax.fori_loop(0, C, body, 0)

    lanes16 = lax.iota(jnp.int32, L) * L
    for g in range(C // L):
        score = jnp.zeros((L,), jnp.float32)
        for col in range(L):
            idx = lanes16 + (g * L * L + col)
            score = score + plsc.load_gather(part_ref, [idx])
        out_ref[pl.ds(out_base + g * L, L)] = score


def _sc_body(head_hbm, relidx_hbm, tail_hbm, ent_hbm, rel_hbm, out_hbm,
             idx_h, idx_r, idx_t, h0, h1, r0, r1, t0, t1, part, out_v,
             sem0, sem1):
    wid = lax.axis_index("s") * NC + lax.axis_index("c")
    base = wid * BPW

    for c in range(NCHUNK):
        pltpu.sync_copy(head_hbm.at[pl.ds(base + c * C, C)], idx_h.at[c])
        pltpu.sync_copy(relidx_hbm.at[pl.ds(base + c * C, C)], idx_r.at[c])
        pltpu.sync_copy(tail_hbm.at[pl.ds(base + c * C, C)], idx_t.at[c])

    hbufs, rbufs, tbufs, sems = (h0, h1), (r0, r1), (t0, t1), (sem0, sem1)
    descs = [None, None]
    for c in range(NCHUNK):
        slot = c % 2
        descs[slot] = (
            pltpu.async_copy(ent_hbm.at[idx_h.at[c]], hbufs[slot], sems[slot]),
            pltpu.async_copy(rel_hbm.at[idx_r.at[c]], rbufs[slot], sems[slot]),
            pltpu.async_copy(ent_hbm.at[idx_t.at[c]], tbufs[slot], sems[slot]),
        )
        if c >= 1:
            ps = (c - 1) % 2
            for dsc in descs[ps]:
                dsc.wait()
            _compute_chunk(hbufs[ps], rbufs[ps], tbufs[ps], part, out_v,
                           (c - 1) * C)
    ls = (NCHUNK - 1) % 2
    for dsc in descs[ls]:
        dsc.wait()
    _compute_chunk(hbufs[ls], rbufs[ls], tbufs[ls], part, out_v,
                   (NCHUNK - 1) * C)

    pltpu.sync_copy(out_v, out_hbm.at[pl.ds(base, BPW)])


def kernel(head, relation, tail, entity_embeddings, relation_embeddings):
    mesh = plsc.VectorSubcoreMesh(core_axis_name="c", subcore_axis_name="s",
                                  num_cores=NC, num_subcores=NS)
    kfn = pl.kernel(
        _sc_body,
        out_type=jax.ShapeDtypeStruct((B,), jnp.float32),
        mesh=mesh,
        scratch_types=[
            pltpu.VMEM((NCHUNK, C), jnp.int32),
            pltpu.VMEM((NCHUNK, C), jnp.int32),
            pltpu.VMEM((NCHUNK, C), jnp.int32),
            pltpu.VMEM((C, D), jnp.float32),
            pltpu.VMEM((C, D), jnp.float32),
            pltpu.VMEM((C, D), jnp.float32),
            pltpu.VMEM((C, D), jnp.float32),
            pltpu.VMEM((C, D), jnp.float32),
            pltpu.VMEM((C, D), jnp.float32),
            pltpu.VMEM((C * L,), jnp.float32),
            pltpu.VMEM((BPW,), jnp.float32),
            pltpu.SemaphoreType.DMA,
            pltpu.SemaphoreType.DMA,
        ],
    )
    return kfn(head, relation, tail, entity_embeddings, relation_embeddings)


# speedup vs baseline: 2.3055x; 2.3055x over previous
import jax
import jax.numpy as jnp
from jax import lax
from jax.experimental import pallas as pl
from jax.experimental.pallas import tpu as pltpu
from jax.experimental.pallas import tpu_sc as plsc

B = 16384
D = 128
L = 16
NC = 2
NS = 16
NW = NC * NS
BPW = B // NW
C = 128
NCHUNK = BPW // C


def _compute_chunk(h_ref, r_ref, t_ref, out_ref, out_base):
    lanes = lax.iota(jnp.int32, L)

    def body(g, carry):
        def inner(l, score):
            e = g * L + l
            acc = jnp.zeros((L,), jnp.float32)
            for j in range(D // L):
                hv = h_ref[e, pl.ds(j * L, L)]
                rv = r_ref[e, pl.ds(j * L, L)]
                tv = t_ref[e, pl.ds(j * L, L)]
                acc = acc + hv * rv * tv
            s = jnp.sum(acc)
            return jnp.where(lanes == l, s, score)

        score = lax.fori_loop(0, L, inner, jnp.zeros((L,), jnp.float32))
        out_ref[pl.ds(out_base + g * L, L)] = score
        return carry

    lax.fori_loop(0, C // L, body, 0)


def _sc_body(head_hbm, relidx_hbm, tail_hbm, ent_hbm, rel_hbm, out_hbm,
             idx_h, idx_r, idx_t, h0, h1, r0, r1, t0, t1, out_v,
             sem0, sem1):
    wid = lax.axis_index("s") * NC + lax.axis_index("c")
    base = wid * BPW

    for c in range(NCHUNK):
        pltpu.sync_copy(head_hbm.at[pl.ds(base + c * C, C)], idx_h.at[c])
        pltpu.sync_copy(relidx_hbm.at[pl.ds(base + c * C, C)], idx_r.at[c])
        pltpu.sync_copy(tail_hbm.at[pl.ds(base + c * C, C)], idx_t.at[c])

    hbufs, rbufs, tbufs, sems = (h0, h1), (r0, r1), (t0, t1), (sem0, sem1)
    descs = [None, None]
    for c in range(NCHUNK):
        slot = c % 2
        descs[slot] = (
            pltpu.async_copy(ent_hbm.at[idx_h.at[c]], hbufs[slot], sems[slot]),
            pltpu.async_copy(rel_hbm.at[idx_r.at[c]], rbufs[slot], sems[slot]),
            pltpu.async_copy(ent_hbm.at[idx_t.at[c]], tbufs[slot], sems[slot]),
        )
        if c >= 1:
            ps = (c - 1) % 2
            for dsc in descs[ps]:
                dsc.wait()
            _compute_chunk(hbufs[ps], rbufs[ps], tbufs[ps], out_v,
                           (c - 1) * C)
    ls = (NCHUNK - 1) % 2
    for dsc in descs[ls]:
        dsc.wait()
    _compute_chunk(hbufs[ls], rbufs[ls], tbufs[ls], out_v,
                   (NCHUNK - 1) * C)

    pltpu.sync_copy(out_v, out_hbm.at[pl.ds(base, BPW)])


def kernel(head, relation, tail, entity_embeddings, relation_embeddings):
    mesh = plsc.VectorSubcoreMesh(core_axis_name="c", subcore_axis_name="s",
                                  num_cores=NC, num_subcores=NS)
    kfn = pl.kernel(
        _sc_body,
        out_type=jax.ShapeDtypeStruct((B,), jnp.float32),
        mesh=mesh,
        compiler_params=pltpu.CompilerParams(needs_layout_passes=False),
        scratch_types=[
            pltpu.VMEM((NCHUNK, C), jnp.int32),
            pltpu.VMEM((NCHUNK, C), jnp.int32),
            pltpu.VMEM((NCHUNK, C), jnp.int32),
            pltpu.VMEM((C, D), jnp.float32),
            pltpu.VMEM((C, D), jnp.float32),
            pltpu.VMEM((C, D), jnp.float32),
            pltpu.VMEM((C, D), jnp.float32),
            pltpu.VMEM((C, D), jnp.float32),
            pltpu.VMEM((C, D), jnp.float32),
            pltpu.VMEM((BPW,), jnp.float32),
            pltpu.SemaphoreType.DMA,
            pltpu.SemaphoreType.DMA,
        ],
    )
    return kfn(head, relation, tail, entity_embeddings, relation_embeddings)
